# baseline (speedup 1.0000x reference)
import functools

import jax
import jax.numpy as jnp
import numpy as np
from jax.experimental import pallas as pl
from jax.experimental.pallas import tpu as pltpu

_ROWS_PER_BLOCK = 16
_COLS_PER_BLOCK = 8192

_TINY = np.float32(np.finfo(np.float32).tiny)
_SCALE = np.float32(np.float32(1.0) - _TINY)


def _threefry2x32(hi, lo):
    ks0 = jnp.uint32(0)
    ks1 = jnp.uint32(42)
    ks2 = jnp.uint32(0x1BD11BDA ^ 42)
    x0 = hi + ks0
    x1 = lo + ks1
    r0 = (13, 15, 26, 6)
    r1 = (17, 29, 16, 24)

    def four_rounds(x0, x1, rots):
        for r in rots:
            x0 = x0 + x1
            x1 = ((x1 << jnp.uint32(r)) | (x1 >> jnp.uint32(32 - r))) ^ x0
        return x0, x1

    for n, (a, b, c) in enumerate(
        ((ks1, ks2, 1), (ks2, ks0, 2), (ks0, ks1, 3), (ks1, ks2, 4), (ks2, ks0, 5))
    ):
        x0, x1 = four_rounds(x0, x1, r0 if n % 2 == 0 else r1)
        x0 = x0 + a
        x1 = x1 + b + jnp.uint32(c)
    return x0 ^ x1


def _sample_kernel(logits_ref, out_ref, max_ref, arg_ref, *, n_cols, n_col_blocks):
    r = pl.program_id(0)
    c = pl.program_id(1)

    @pl.when(c == 0)
    def _init():
        max_ref[...] = jnp.full_like(max_ref, -jnp.inf)
        arg_ref[...] = jnp.zeros_like(arg_ref)

    block = logits_ref[...]
    shape = block.shape
    row_ids = r * _ROWS_PER_BLOCK + jax.lax.broadcasted_iota(jnp.int32, shape, 0)
    col_ids = c * _COLS_PER_BLOCK + jax.lax.broadcasted_iota(jnp.int32, shape, 1)
    lin = (row_ids * n_cols + col_ids).astype(jnp.uint32)

    bits = _threefry2x32(jnp.zeros_like(lin), lin)
    fb = (bits >> jnp.uint32(9)) | jnp.uint32(0x3F800000)
    u = jax.lax.bitcast_convert_type(fb, jnp.float32) - jnp.float32(1.0)
    u = jnp.maximum(_TINY, u * _SCALE + _TINY)
    g = -jnp.log(-jnp.log(u))
    val = g + block
    val = jnp.where(col_ids < n_cols, val, -jnp.inf)

    bmax = jnp.max(val, axis=1, keepdims=True)
    ids = jnp.where(val == bmax, col_ids, jnp.int32(0x7FFFFFFF))
    bidx = jnp.min(ids, axis=1, keepdims=True)

    better = bmax > max_ref[...]
    max_ref[...] = jnp.where(better, bmax, max_ref[...])
    arg_ref[...] = jnp.where(better, bidx, arg_ref[...])

    @pl.when(c == n_col_blocks - 1)
    def _finalize():
        out_ref[...] = arg_ref[...]


def kernel(logits):
    n_rows, n_cols = logits.shape
    n_row_blocks = pl.cdiv(n_rows, _ROWS_PER_BLOCK)
    n_col_blocks = pl.cdiv(n_cols, _COLS_PER_BLOCK)

    out = pl.pallas_call(
        functools.partial(
            _sample_kernel, n_cols=n_cols, n_col_blocks=n_col_blocks
        ),
        grid=(n_row_blocks, n_col_blocks),
        in_specs=[
            pl.BlockSpec((_ROWS_PER_BLOCK, _COLS_PER_BLOCK), lambda r, c: (r, c)),
        ],
        out_specs=pl.BlockSpec((_ROWS_PER_BLOCK, 1), lambda r, c: (r, 0)),
        out_shape=jax.ShapeDtypeStruct((n_rows, 1), jnp.int32),
        scratch_shapes=[
            pltpu.VMEM((_ROWS_PER_BLOCK, 1), jnp.float32),
            pltpu.VMEM((_ROWS_PER_BLOCK, 1), jnp.int32),
        ],
    )(logits)
    return out[:, 0]


# speedup vs baseline: 1.5462x; 1.5462x over previous
import functools

import jax
import jax.numpy as jnp
import numpy as np
from jax.experimental import pallas as pl
from jax.experimental.pallas import tpu as pltpu

_ROWS_PER_BLOCK = 16
_COLS_PER_BLOCK = 8192
_LANES = 128

_TINY = np.float32(np.finfo(np.float32).tiny)
_NEG_INF = np.float32(-np.inf)
_INT_MAX = np.int32(0x7FFFFFFF)

_KS0 = 0
_KS1 = 42
_KS2 = 0x1BD11BDA ^ _KS0 ^ _KS1
_ROT0 = (13, 15, 26, 6)
_ROT1 = (17, 29, 16, 24)


def _rotl(x, r):
    return (x << jnp.uint32(r)) | (x >> jnp.uint32(32 - r))


def _threefry_bits(lin):
    x1 = lin + jnp.uint32(_KS1)
    x0 = x1
    x1 = _rotl(x1, _ROT0[0]) ^ x0
    for r in _ROT0[1:]:
        x0 = x0 + x1
        x1 = _rotl(x1, r) ^ x0
    x0 = x0 + jnp.uint32(_KS1)
    x1 = x1 + jnp.uint32((_KS2 + 1) & 0xFFFFFFFF)
    for r in _ROT1:
        x0 = x0 + x1
        x1 = _rotl(x1, r) ^ x0
    x0 = x0 + jnp.uint32(_KS2)
    x1 = x1 + jnp.uint32(_KS0 + 2)
    for r in _ROT0:
        x0 = x0 + x1
        x1 = _rotl(x1, r) ^ x0
    x1 = x1 + jnp.uint32(_KS1 + 3)
    for r in _ROT1:
        x0 = x0 + x1
        x1 = _rotl(x1, r) ^ x0
    x0 = x0 + jnp.uint32(_KS1)
    x1 = x1 + jnp.uint32((_KS2 + 4) & 0xFFFFFFFF)
    for r in _ROT0:
        x0 = x0 + x1
        x1 = _rotl(x1, r) ^ x0
    x0 = x0 + jnp.uint32(_KS2)
    x1 = x1 + jnp.uint32(_KS0 + 5)
    return x0 ^ x1


def _sample_kernel(logits_ref, out_ref, max_ref, arg_ref, *, n_cols, n_col_blocks):
    r = pl.program_id(0)
    c = pl.program_id(1)
    rows = _ROWS_PER_BLOCK

    @pl.when(c == 0)
    def _init():
        max_ref[...] = jnp.full_like(max_ref, _NEG_INF)
        arg_ref[...] = jnp.full_like(arg_ref, _INT_MAX)

    shape = (rows, _LANES)
    lane = jax.lax.broadcasted_iota(jnp.int32, shape, 1)
    row = r * rows + jax.lax.broadcasted_iota(jnp.int32, shape, 0)
    row_base = row * n_cols
    col0 = c * _COLS_PER_BLOCK + lane

    run_max = max_ref[...]
    run_arg = arg_ref[...]
    for j in range(_COLS_PER_BLOCK // _LANES):
        col = col0 + j * _LANES
        lin = (row_base + col).astype(jnp.uint32)
        bits = _threefry_bits(lin)
        fb = (bits >> jnp.uint32(9)) | jnp.uint32(0x3F800000)
        u = jax.lax.bitcast_convert_type(fb, jnp.float32) - jnp.float32(1.0)
        u = jnp.maximum(_TINY, u + _TINY)
        g = -jnp.log(-jnp.log(u))
        val = g + logits_ref[:, j * _LANES : (j + 1) * _LANES]
        val = jnp.where(col < n_cols, val, _NEG_INF)
        upd = val > run_max
        run_max = jnp.where(upd, val, run_max)
        run_arg = jnp.where(upd, col, run_arg)
    max_ref[...] = run_max
    arg_ref[...] = run_arg

    @pl.when(c == n_col_blocks - 1)
    def _finalize():
        m = max_ref[...]
        row_max = jnp.max(m, axis=1, keepdims=True)
        ids = jnp.where(m == row_max, arg_ref[...], _INT_MAX)
        out_ref[...] = jnp.min(ids, axis=1, keepdims=True)


def kernel(logits):
    n_rows, n_cols = logits.shape
    n_row_blocks = pl.cdiv(n_rows, _ROWS_PER_BLOCK)
    n_col_blocks = pl.cdiv(n_cols, _COLS_PER_BLOCK)

    out = pl.pallas_call(
        functools.partial(
            _sample_kernel, n_cols=n_cols, n_col_blocks=n_col_blocks
        ),
        grid=(n_row_blocks, n_col_blocks),
        in_specs=[
            pl.BlockSpec((_ROWS_PER_BLOCK, _COLS_PER_BLOCK), lambda r, c: (r, c)),
        ],
        out_specs=pl.BlockSpec((_ROWS_PER_BLOCK, 1), lambda r, c: (r, 0)),
        out_shape=jax.ShapeDtypeStruct((n_rows, 1), jnp.int32),
        scratch_shapes=[
            pltpu.VMEM((_ROWS_PER_BLOCK, _LANES), jnp.float32),
            pltpu.VMEM((_ROWS_PER_BLOCK, _LANES), jnp.int32),
        ],
    )(logits)
    return out[:, 0]


# speedup vs baseline: 1.5762x; 1.0194x over previous
import functools

import jax
import jax.numpy as jnp
import numpy as np
from jax.experimental import pallas as pl
from jax.experimental.pallas import tpu as pltpu

_ROWS_PER_BLOCK = 32
_COLS_PER_BLOCK = 8192
_LANES = 128

_TINY = np.float32(np.finfo(np.float32).tiny)
_NEG_INF = np.float32(-np.inf)
_INT_MAX = np.int32(0x7FFFFFFF)

_KS0 = 0
_KS1 = 42
_KS2 = 0x1BD11BDA ^ _KS0 ^ _KS1
_ROT0 = (13, 15, 26, 6)
_ROT1 = (17, 29, 16, 24)


def _rotl(x, r):
    return (x << jnp.uint32(r)) | (x >> jnp.uint32(32 - r))


def _threefry_bits(lin):
    x1 = lin + jnp.uint32(_KS1)
    x0 = x1
    x1 = _rotl(x1, _ROT0[0]) ^ x0
    for r in _ROT0[1:]:
        x0 = x0 + x1
        x1 = _rotl(x1, r) ^ x0
    x0 = x0 + jnp.uint32(_KS1)
    x1 = x1 + jnp.uint32((_KS2 + 1) & 0xFFFFFFFF)
    for r in _ROT1:
        x0 = x0 + x1
        x1 = _rotl(x1, r) ^ x0
    x0 = x0 + jnp.uint32(_KS2)
    x1 = x1 + jnp.uint32(_KS0 + 2)
    for r in _ROT0:
        x0 = x0 + x1
        x1 = _rotl(x1, r) ^ x0
    x1 = x1 + jnp.uint32(_KS1 + 3)
    for r in _ROT1:
        x0 = x0 + x1
        x1 = _rotl(x1, r) ^ x0
    x0 = x0 + jnp.uint32(_KS1)
    x1 = x1 + jnp.uint32((_KS2 + 4) & 0xFFFFFFFF)
    for r in _ROT0:
        x0 = x0 + x1
        x1 = _rotl(x1, r) ^ x0
    x0 = x0 + jnp.uint32(_KS2)
    x1 = x1 + jnp.uint32(_KS0 + 5)
    return x0 ^ x1


def _sample_kernel(logits_ref, out_ref, max_ref, arg_ref, *, n_cols, n_col_blocks):
    r = pl.program_id(0)
    c = pl.program_id(1)
    rows = _ROWS_PER_BLOCK

    @pl.when(c == 0)
    def _init():
        max_ref[...] = jnp.full_like(max_ref, _NEG_INF)
        arg_ref[...] = jnp.full_like(arg_ref, _INT_MAX)

    shape = (rows, _LANES)
    lane = jax.lax.broadcasted_iota(jnp.int32, shape, 1)
    row = r * rows + jax.lax.broadcasted_iota(jnp.int32, shape, 0)
    row_base = row * n_cols
    lin0 = row_base + c * _COLS_PER_BLOCK + lane
    bound = row_base + n_cols

    run_max = max_ref[...]
    run_arg = arg_ref[...]
    for j in range(_COLS_PER_BLOCK // _LANES):
        lin = lin0 + j * _LANES
        bits = _threefry_bits(lin.astype(jnp.uint32))
        fb = (bits >> jnp.uint32(9)) | jnp.uint32(0x3F800000)
        u = jax.lax.bitcast_convert_type(fb, jnp.float32) - jnp.float32(1.0)
        u = jnp.maximum(u, _TINY)
        g = -jnp.log(-jnp.log(u))
        val = g + logits_ref[:, j * _LANES : (j + 1) * _LANES]
        val = jnp.where(lin < bound, val, _NEG_INF)
        upd = val > run_max
        run_max = jnp.where(upd, val, run_max)
        run_arg = jnp.where(upd, lin, run_arg)
    max_ref[...] = run_max
    arg_ref[...] = run_arg

    @pl.when(c == n_col_blocks - 1)
    def _finalize():
        m = max_ref[...]
        row_max = jnp.max(m, axis=1, keepdims=True)
        ids = jnp.where(m == row_max, arg_ref[...], _INT_MAX)
        out_ref[...] = jnp.min(ids, axis=1, keepdims=True) - row_base[:, :1]


def kernel(logits):
    n_rows, n_cols = logits.shape
    n_row_blocks = pl.cdiv(n_rows, _ROWS_PER_BLOCK)
    n_col_blocks = pl.cdiv(n_cols, _COLS_PER_BLOCK)

    out = pl.pallas_call(
        functools.partial(
            _sample_kernel, n_cols=n_cols, n_col_blocks=n_col_blocks
        ),
        grid=(n_row_blocks, n_col_blocks),
        in_specs=[
            pl.BlockSpec((_ROWS_PER_BLOCK, _COLS_PER_BLOCK), lambda r, c: (r, c)),
        ],
        out_specs=pl.BlockSpec((_ROWS_PER_BLOCK, 1), lambda r, c: (r, 0)),
        out_shape=jax.ShapeDtypeStruct((n_rows, 1), jnp.int32),
        scratch_shapes=[
            pltpu.VMEM((_ROWS_PER_BLOCK, _LANES), jnp.float32),
            pltpu.VMEM((_ROWS_PER_BLOCK, _LANES), jnp.int32),
        ],
    )(logits)
    return out[:, 0]
